# baseline (speedup 1.0000x reference)
import jax
import jax.numpy as jnp
from jax.experimental import pallas as pl
from jax.experimental.pallas import tpu as pltpu

_NE = 1024
_D = 64
_B = 32
_L = 576
_N = _B * _L
_CC = 0.1
_BB = 8
_G = _B // _BB


def _vq_body(x_ref, e_ref, e2_ref, jcol_ref, out_ref, idx_ref, loss_ref,
             perp_ref, cnt_ref, kl_ref):
    g = pl.program_id(0)

    @pl.when(g == 0)
    def _init():
        cnt_ref[...] = jnp.zeros_like(cnt_ref)
        kl_ref[...] = jnp.zeros_like(kl_ref)

    e = e_ref[...]
    e2 = e2_ref[...]
    jcol = jcol_ref[...]
    em2 = e * -2.0
    for s in range(_BB):
        x = x_ref[s]
        x2 = jnp.sum(x * x, axis=0)
        scores_m2 = jax.lax.dot_general(
            em2, x, dimension_numbers=(((1,), (0,)), ((), ())),
            preferred_element_type=jnp.float32)
        dist = (x2[None, :] + e2) + scores_m2
        m = jnp.min(dist, axis=0)
        idx_f = jnp.min(jnp.where(dist == m[None, :], jcol, float(_NE)),
                        axis=0)
        idx = idx_f.astype(jnp.int32)

        onehot = (jcol == idx_f[None, :]).astype(jnp.float32)
        q = jax.lax.dot_general(
            e, onehot, dimension_numbers=(((0,), (0,)), ((), ())),
            preferred_element_type=jnp.float32)

        out_ref[s] = x + (q - x)
        idx_ref[s, 0] = idx
        cnt_ref[...] += jnp.sum(onehot, axis=1, keepdims=True)

        sm_x = jax.nn.softmax(x, axis=0)
        sm_q = jax.nn.softmax(q, axis=0)
        kl_ref[...] += jnp.sum(sm_x * (jnp.log(sm_x) - sm_q)).reshape(1, 1)

    @pl.when(g == _G - 1)
    def _fin():
        loss_ref[...] = _CC * kl_ref[...] / _B
        p = cnt_ref[...] / _N
        perp_ref[...] = jnp.exp(-jnp.sum(p * jnp.log(p + 1e-10))).reshape(1, 1)


def _vq_call(inputs, embedding_weight, e2, jcol):
    return pl.pallas_call(
        _vq_body,
        grid=(_G,),
        in_specs=[
            pl.BlockSpec((_BB, _D, _L), lambda g: (g, 0, 0)),
            pl.BlockSpec((_NE, _D), lambda g: (0, 0)),
            pl.BlockSpec((_NE, 1), lambda g: (0, 0)),
            pl.BlockSpec((_NE, 1), lambda g: (0, 0)),
        ],
        out_specs=[
            pl.BlockSpec((_BB, _D, _L), lambda g: (g, 0, 0)),
            pl.BlockSpec((_BB, 1, _L), lambda g: (g, 0, 0)),
            pl.BlockSpec((1, 1), lambda g: (0, 0)),
            pl.BlockSpec((1, 1), lambda g: (0, 0)),
        ],
        out_shape=[
            jax.ShapeDtypeStruct((_B, _D, _L), jnp.float32),
            jax.ShapeDtypeStruct((_B, 1, _L), jnp.int32),
            jax.ShapeDtypeStruct((1, 1), jnp.float32),
            jax.ShapeDtypeStruct((1, 1), jnp.float32),
        ],
        scratch_shapes=[
            pltpu.VMEM((_NE, 1), jnp.float32),
            pltpu.VMEM((1, 1), jnp.float32),
        ],
    )(inputs, embedding_weight, e2, jcol)


def kernel(inputs, embedding_weight):
    e2 = jnp.sum(embedding_weight ** 2, axis=1)[:, None]
    jcol = jnp.arange(_NE, dtype=jnp.float32)[:, None]
    out, idx, loss, perp = _vq_call(inputs, embedding_weight, e2, jcol)
    return (out, loss[0, 0], perp[0, 0], embedding_weight,
            idx.reshape(_N, 1))
